# baseline (speedup 1.0000x reference)
import functools

import jax
import jax.numpy as jnp
from jax import lax
from jax.experimental import pallas as pl
from jax.experimental.pallas import tpu as pltpu
from jax.experimental.pallas import tpu_sc as plsc

_B = 16384
_D = 32
_INFO = plsc.get_sparse_core_info()
_NC, _NS, _L = _INFO.num_cores, _INFO.num_subcores, _INFO.num_lanes
_NW = _NC * _NS
_BPW = _B // _NW


def _make_embed():
    mesh = plsc.VectorSubcoreMesh(core_axis_name="c", subcore_axis_name="s")

    @functools.partial(
        pl.kernel,
        mesh=mesh,
        out_type=jax.ShapeDtypeStruct((_B, _D), jnp.float32),
        scratch_types=[
            pltpu.VMEM((_BPW,), jnp.int32),
            pltpu.VMEM((_BPW, _D), jnp.float32),
            pltpu.SemaphoreType.DMA,
        ],
        compiler_params=pltpu.CompilerParams(use_tc_tiling_on_sc=False),
    )
    def embed(labels_hbm, table_hbm, out_hbm, idx_v, rows_v, sem):
        wid = lax.axis_index("s") * _NC + lax.axis_index("c")
        base = wid * _BPW
        pltpu.sync_copy(labels_hbm.at[pl.ds(base, _BPW)], idx_v)

        def body(i, carry):
            sl = pl.ds(i * _L, _L)
            idx_v[sl] = idx_v[sl] + 1
            return carry

        lax.fori_loop(0, _BPW // _L, body, 0)
        pltpu.async_copy(table_hbm.at[idx_v], rows_v, sem).wait()
        pltpu.sync_copy(rows_v, out_hbm.at[pl.ds(base, _BPW)])

    return embed


_embed = _make_embed()


def kernel(class_labels, table):
    return _embed(class_labels, table)


# speedup vs baseline: 1.6604x; 1.6604x over previous
import functools

import jax
import jax.numpy as jnp
from jax import lax
from jax.experimental import pallas as pl
from jax.experimental.pallas import tpu as pltpu
from jax.experimental.pallas import tpu_sc as plsc

_B = 16384
_D = 32
_INFO = plsc.get_sparse_core_info()
_NC, _NS, _L = _INFO.num_cores, _INFO.num_subcores, _INFO.num_lanes
_NW = _NC * _NS
_BPW = _B // _NW
_NG = _BPW // _L


def _make_embed():
    mesh = plsc.VectorSubcoreMesh(core_axis_name="c", subcore_axis_name="s")

    @functools.partial(
        pl.kernel,
        mesh=mesh,
        out_type=jax.ShapeDtypeStruct((_B, _D), jnp.float32),
        scratch_types=[
            pltpu.VMEM((_BPW,), jnp.int32),
            pltpu.VMEM((_BPW, _D), jnp.float32),
            pltpu.SemaphoreType.DMA,
        ],
    )
    def embed(labels_hbm, table_hbm, out_hbm, idx_v, rows_v, sem):
        wid = lax.axis_index("s") * _NC + lax.axis_index("c")
        base = wid * _BPW
        pltpu.sync_copy(labels_hbm.at[pl.ds(base, _BPW)], idx_v)

        def fire(g, carry):
            idx16 = idx_v[pl.ds(g * _L, _L)] + 1
            for j in range(_L):
                r = idx16[j]
                pltpu.async_copy(
                    table_hbm.at[pl.ds(r, 1)],
                    rows_v.at[pl.ds(g * _L + j, 1)],
                    sem,
                )
            return carry

        lax.fori_loop(0, _NG, fire, 0)

        def drain(i, carry):
            pltpu.make_async_copy(
                table_hbm.at[pl.ds(0, 1)], rows_v.at[pl.ds(i, 1)], sem
            ).wait()
            return carry

        lax.fori_loop(0, _BPW, drain, 0)
        pltpu.sync_copy(rows_v, out_hbm.at[pl.ds(base, _BPW)])

    return embed


_embed = _make_embed()


def kernel(class_labels, table):
    return _embed(class_labels, table)


# speedup vs baseline: 3.5898x; 2.1620x over previous
import functools

import jax
import jax.numpy as jnp
from jax import lax
from jax.experimental import pallas as pl
from jax.experimental.pallas import tpu as pltpu
from jax.experimental.pallas import tpu_sc as plsc

_B = 16384
_D = 32
_INFO = plsc.get_sparse_core_info()
_NC, _NS, _L = _INFO.num_cores, _INFO.num_subcores, _INFO.num_lanes
_NW = _NC * _NS
_BPW = _B // _NW
_NG = _BPW // _L
_BLK = 128


def _make_embed():
    mesh = plsc.VectorSubcoreMesh(core_axis_name="c", subcore_axis_name="s")

    @functools.partial(
        pl.kernel,
        mesh=mesh,
        out_type=jax.ShapeDtypeStruct((_D, _B), jnp.float32),
        scratch_types=[
            pltpu.VMEM((_BPW,), jnp.int32),
            pltpu.VMEM((_L, _D, _BLK), jnp.float32),
            pltpu.VMEM((_D, _BPW), jnp.float32),
            pltpu.SemaphoreType.DMA,
        ],
        compiler_params=pltpu.CompilerParams(needs_layout_passes=False),
    )
    def embed(labels_hbm, tablet_hbm, outt_hbm, idx_v, blk_v, cols_v, sem):
        wid = lax.axis_index("s") * _NC + lax.axis_index("c")
        base = wid * _BPW
        pltpu.sync_copy(labels_hbm.at[pl.ds(base, _BPW)], idx_v)

        d_lo = lax.iota(jnp.int32, _L)
        d_hi = d_lo + _L

        def group(g, carry):
            idx16 = idx_v[pl.ds(g * _L, _L)] + 1
            blk16 = lax.shift_right_logical(idx16, 7)
            off16 = lax.bitwise_and(idx16, 127)
            for j in range(_L):
                c0 = pl.multiple_of(blk16[j] * _BLK, _BLK)
                pltpu.async_copy(
                    tablet_hbm.at[:, pl.ds(c0, _BLK)], blk_v.at[j], sem
                )
            for j in range(_L):
                pltpu.make_async_copy(
                    tablet_hbm.at[:, pl.ds(0, _BLK)], blk_v.at[j], sem
                ).wait()
            for j in range(_L):
                jb = jnp.full((_L,), j, jnp.int32)
                ob = jnp.full((_L,), off16[j], jnp.int32)
                pos = jnp.full((_L,), g * _L + j, jnp.int32)
                v_lo = plsc.load_gather(blk_v, [jb, d_lo, ob])
                v_hi = plsc.load_gather(blk_v, [jb, d_hi, ob])
                plsc.store_scatter(cols_v, [d_lo, pos], v_lo)
                plsc.store_scatter(cols_v, [d_hi, pos], v_hi)
            return carry

        lax.fori_loop(0, _NG, group, 0)
        pltpu.sync_copy(cols_v, outt_hbm.at[:, pl.ds(base, _BPW)])

    return embed


_embed = _make_embed()


def kernel(class_labels, table):
    outt = _embed(class_labels, table.T)
    return outt.T
